# baseline (speedup 1.0000x reference)
import jax
import jax.numpy as jnp
from jax import lax
from jax.experimental import pallas as pl
from jax.experimental.pallas import tpu as pltpu
from jax.experimental.pallas import tpu_sc as plsc

NUM_EMBEDDINGS = 1000000
EMBEDDING_DIM = 32
BATCH = 16384
N_FIELDS = 26

NC = 2
NS = 16
NW = NC * NS

TOTAL = BATCH * N_FIELDS
PER_W = TOTAL // NW
CHUNK = 128
NCHUNK = PER_W // CHUNK

WPR = EMBEDDING_DIM // 2
ODIM = EMBEDDING_DIM


def _body(idx_hbm, tab_hbm, scale_hbm, out_hbm, idx_v, rows_v, out_v,
          scale_v, sem):
    wid = lax.axis_index("s") * NC + lax.axis_index("c")
    base = wid * PER_W

    pltpu.sync_copy(scale_hbm, scale_v)
    scale = scale_v[...]

    lane = lax.iota(jnp.int32, 16)
    col_e = lane * 2

    def chunk_body(ci, _):
        cbase = base + ci * CHUNK

        pltpu.async_copy(
            idx_hbm.at[pl.ds(cbase, CHUNK)], idx_v, sem).wait()
        pltpu.async_copy(tab_hbm.at[idx_v], rows_v, sem).wait()

        def row_body(j, _):
            w = rows_v[j, :]
            lo = ((w << jnp.uint32(16)) & jnp.uint32(0x80000000)) | (
                (w << jnp.uint32(13)) & jnp.uint32(0x0FFFE000))
            hi = (w & jnp.uint32(0x80000000)) | (
                (w >> jnp.uint32(3)) & jnp.uint32(0x0FFFE000))
            ev = lax.bitcast_convert_type(lo, jnp.float32) * scale
            od = lax.bitcast_convert_type(hi, jnp.float32) * scale
            addr_e = j * ODIM + col_e
            plsc.store_scatter(out_v, [addr_e], ev)
            plsc.store_scatter(out_v, [addr_e + 1], od)
            return 0

        lax.fori_loop(0, CHUNK, row_body, 0)

        pltpu.async_copy(
            out_v, out_hbm.at[pl.ds(cbase * ODIM, CHUNK * ODIM)], sem
        ).wait()
        return 0

    lax.fori_loop(0, NCHUNK, chunk_body, 0)


@jax.jit
def _run(x_flat, tab_u32, scale16):
    mesh = plsc.VectorSubcoreMesh(core_axis_name="c", subcore_axis_name="s")
    f = pl.kernel(
        _body,
        mesh=mesh,
        compiler_params=pltpu.CompilerParams(
            needs_layout_passes=False, use_tc_tiling_on_sc=False),
        out_type=jax.ShapeDtypeStruct((TOTAL * ODIM,), jnp.float32),
        scratch_types=[
            pltpu.VMEM((CHUNK,), jnp.int32),
            pltpu.VMEM((CHUNK, WPR), jnp.uint32),
            pltpu.VMEM((CHUNK * ODIM,), jnp.float32),
            pltpu.VMEM((16,), jnp.float32),
            pltpu.SemaphoreType.DMA,
        ],
    )
    return f(x_flat, tab_u32, scale16)


def kernel(x, weight_quant, c):
    x_flat = x.reshape(TOTAL)
    tab_u32 = lax.bitcast_convert_type(
        weight_quant.reshape(NUM_EMBEDDINGS, WPR, 2), jnp.uint32)
    scale = jnp.float32(2.0 ** 112) / c
    scale16 = jnp.broadcast_to(scale, (16,))
    out = _run(x_flat, tab_u32, scale16)
    return out.reshape(BATCH, N_FIELDS, EMBEDDING_DIM)


# speedup vs baseline: 1.0710x; 1.0710x over previous
import jax
import jax.numpy as jnp
from jax import lax
from jax.experimental import pallas as pl
from jax.experimental.pallas import tpu as pltpu
from jax.experimental.pallas import tpu_sc as plsc

NUM_EMBEDDINGS = 1000000
EMBEDDING_DIM = 32
BATCH = 16384
N_FIELDS = 26

NC = 2
NS = 16
NW = NC * NS

TOTAL = BATCH * N_FIELDS
PER_W = TOTAL // NW
CHUNK = 1024
NCHUNK = PER_W // CHUNK

WPR = EMBEDDING_DIM // 2
ODIM = EMBEDDING_DIM


def _body(idx_hbm, tab_hbm, scale_hbm, out_hbm, idx_v, rows_v, out_v,
          scale_v, sem):
    wid = lax.axis_index("s") * NC + lax.axis_index("c")
    base = wid * PER_W

    pltpu.sync_copy(scale_hbm, scale_v)
    scale = scale_v[...]

    lane = lax.iota(jnp.int32, 16)
    col_e = lane * 2

    def chunk_body(ci, _):
        cbase = base + ci * CHUNK

        pltpu.async_copy(
            idx_hbm.at[pl.ds(cbase, CHUNK)], idx_v, sem).wait()
        pltpu.async_copy(tab_hbm.at[idx_v], rows_v, sem).wait()

        def row_body(j, _):
            w = rows_v[j, :]
            lo = ((w << jnp.uint32(16)) & jnp.uint32(0x80000000)) | (
                (w << jnp.uint32(13)) & jnp.uint32(0x0FFFE000))
            hi = (w & jnp.uint32(0x80000000)) | (
                (w >> jnp.uint32(3)) & jnp.uint32(0x0FFFE000))
            ev = lax.bitcast_convert_type(lo, jnp.float32) * scale
            od = lax.bitcast_convert_type(hi, jnp.float32) * scale
            addr_e = j * ODIM + col_e
            plsc.store_scatter(out_v, [addr_e], ev)
            plsc.store_scatter(out_v, [addr_e + 1], od)
            return 0

        lax.fori_loop(0, CHUNK, row_body, 0)

        pltpu.async_copy(
            out_v, out_hbm.at[pl.ds(cbase * ODIM, CHUNK * ODIM)], sem
        ).wait()
        return 0

    lax.fori_loop(0, NCHUNK, chunk_body, 0)


@jax.jit
def _run(x_flat, tab_u32, scale16):
    mesh = plsc.VectorSubcoreMesh(core_axis_name="c", subcore_axis_name="s")
    f = pl.kernel(
        _body,
        mesh=mesh,
        compiler_params=pltpu.CompilerParams(
            needs_layout_passes=False, use_tc_tiling_on_sc=False),
        out_type=jax.ShapeDtypeStruct((TOTAL * ODIM,), jnp.float32),
        scratch_types=[
            pltpu.VMEM((CHUNK,), jnp.int32),
            pltpu.VMEM((CHUNK, WPR), jnp.uint32),
            pltpu.VMEM((CHUNK * ODIM,), jnp.float32),
            pltpu.VMEM((16,), jnp.float32),
            pltpu.SemaphoreType.DMA,
        ],
    )
    return f(x_flat, tab_u32, scale16)


def kernel(x, weight_quant, c):
    x_flat = x.reshape(TOTAL)
    tab_u32 = lax.bitcast_convert_type(
        weight_quant.reshape(NUM_EMBEDDINGS, WPR, 2), jnp.uint32)
    scale = jnp.float32(2.0 ** 112) / c
    scale16 = jnp.broadcast_to(scale, (16,))
    out = _run(x_flat, tab_u32, scale16)
    return out.reshape(BATCH, N_FIELDS, EMBEDDING_DIM)


# speedup vs baseline: 1.6769x; 1.5658x over previous
import jax
import jax.numpy as jnp
from jax import lax
from jax.experimental import pallas as pl
from jax.experimental.pallas import tpu as pltpu
from jax.experimental.pallas import tpu_sc as plsc

NUM_EMBEDDINGS = 1000000
EMBEDDING_DIM = 32
BATCH = 16384
N_FIELDS = 26

NC = 2
NS = 16
NW = NC * NS

TC_TILES = 7813
RPAD = TC_TILES * 128

LT = BATCH // 128
UNITS = N_FIELDS * LT
PER_W = UNITS // NW
UPC = 8
NCHUNK = PER_W // UPC
CLOOK = UPC * 128

WPR = EMBEDDING_DIM // 2
UBLK = 4 * 1024

_CP = pltpu.CompilerParams(
    needs_layout_passes=False, use_tc_tiling_on_sc=False)
_MESH = dict(core_axis_name="c", subcore_axis_name="s")


def _detile_body(tab5_hbm, out_hbm, in_v, out_v, sem_i, sem_o):
    wid = lax.axis_index("s") * NC + lax.axis_index("c")
    start = wid * 244 + jnp.minimum(wid, 5)
    cnt = jnp.where(wid < 5, 245, 244)

    lane16 = lax.iota(jnp.int32, 16) * 16

    def tc_body(tc, _):
        pltpu.async_copy(tab5_hbm.at[:, tc, :], in_v, sem_i).wait()
        for tr in range(4):
            for s2 in range(4):
                for m in range(8):
                    v = in_v[tr, pl.ds(s2 * 256 + m * 32, 32)]
                    w = plsc.bitcast(v, jnp.int32)
                    plsc.store_scatter(
                        out_v, [lane16 + (m * 256 + tr * 4 + s2)], w)
        pltpu.async_copy(
            out_v, out_hbm.at[pl.ds(tc * (128 * WPR), 128 * WPR)],
            sem_o).wait()
        return 0

    lax.fori_loop(start, start + cnt, tc_body, 0)


def _gather_body(idx_hbm, tab_hbm, scale_hbm, out_hbm, idx_v, rows_v, out_v,
                 scale_v, sem_i, sem_g, sem_o):
    wid = lax.axis_index("s") * NC + lax.axis_index("c")
    ubase = wid * PER_W

    pltpu.sync_copy(scale_hbm, scale_v)
    scale = scale_v[...]

    lane = lax.iota(jnp.int32, 16)
    e_even = lane * 2
    evec_e = (e_even // 8) * 1024 + (e_even % 8) * 128
    e_odd = e_even + 1
    evec_o = (e_odd // 8) * 1024 + (e_odd % 8) * 128

    def chunk_body(ci, _):
        u0 = ubase + ci * UPC

        icopies = []
        for g in range(UPC):
            f = (u0 + g) // LT
            lt = (u0 + g) % LT
            icopies.append(pltpu.async_copy(
                idx_hbm.at[f, pl.ds(lt * 128, 128)],
                idx_v.at[pl.ds(g * 128, 128)], sem_i))
        for cp in icopies:
            cp.wait()

        pltpu.async_copy(tab_hbm.at[idx_v], rows_v, sem_g).wait()

        def row_body(j, _):
            w = plsc.bitcast(rows_v[j, :], jnp.int32)
            lo = ((w << 16) & jnp.int32(-0x80000000)) | (
                (w << 13) & jnp.int32(0x0FFFE000))
            hi = (w & jnp.int32(-0x80000000)) | (
                lax.shift_right_logical(w, 3) & jnp.int32(0x0FFFE000))
            ev = lax.bitcast_convert_type(lo, jnp.float32) * scale
            od = lax.bitcast_convert_type(hi, jnp.float32) * scale
            base = (j // 128) * UBLK + (j % 128)
            plsc.store_scatter(out_v, [base + evec_e], ev)
            plsc.store_scatter(out_v, [base + evec_o], od)
            return 0

        lax.fori_loop(0, CLOOK, row_body, 0)

        ocopies = []
        for g in range(UPC):
            f = (u0 + g) // LT
            lt = (u0 + g) % LT
            for st in range(4):
                ocopies.append(pltpu.async_copy(
                    out_v.at[pl.ds(g * UBLK + st * 1024, 1024)],
                    out_hbm.at[f, st, lt, :], sem_o))
        for cp in ocopies:
            cp.wait()
        return 0

    lax.fori_loop(0, NCHUNK, chunk_body, 0)


@jax.jit
def _run(xt, tab, scale16):
    gather = pl.kernel(
        _gather_body,
        mesh=plsc.VectorSubcoreMesh(**_MESH),
        compiler_params=_CP,
        out_type=jax.ShapeDtypeStruct((N_FIELDS, 4, LT, 1024), jnp.float32),
        scratch_types=[
            pltpu.VMEM((CLOOK,), jnp.int32),
            pltpu.VMEM((CLOOK, EMBEDDING_DIM), jnp.float16),
            pltpu.VMEM((UPC * UBLK,), jnp.float32),
            pltpu.VMEM((16,), jnp.float32),
            pltpu.SemaphoreType.DMA,
            pltpu.SemaphoreType.DMA,
            pltpu.SemaphoreType.DMA,
        ],
    )
    return gather(xt, tab, scale16)


def kernel(x, weight_quant, c):
    xt = x.T
    scale = jnp.float32(2.0 ** 112) / c
    scale16 = jnp.broadcast_to(scale, (16,))
    out = _run(xt, weight_quant, scale16)
    out = out.reshape(N_FIELDS, 4, LT, 8, 128)
    out = out.transpose(2, 4, 0, 1, 3)
    return out.reshape(BATCH, N_FIELDS, EMBEDDING_DIM)


# speedup vs baseline: 1.7352x; 1.0347x over previous
import jax
import jax.numpy as jnp
from jax import lax
from jax.experimental import pallas as pl
from jax.experimental.pallas import tpu as pltpu
from jax.experimental.pallas import tpu_sc as plsc

NUM_EMBEDDINGS = 1000000
EMBEDDING_DIM = 32
BATCH = 16384
N_FIELDS = 26

NC = 2
NS = 16
NW = NC * NS

LT = BATCH // 128
UNITS = N_FIELDS * LT
PER_W = UNITS // NW
UPC = 8
NCHUNK = PER_W // UPC
CLOOK = UPC * 128

WPR = EMBEDDING_DIM // 2
STBLK = UPC * 1024

_CP = pltpu.CompilerParams(
    needs_layout_passes=False, use_tc_tiling_on_sc=False)
_MESH = dict(core_axis_name="c", subcore_axis_name="s")


def _gather_body(idx_hbm, tab_hbm, scale_hbm, out_hbm,
                 idx_a, idx_b, rows_a, rows_b, out_a, out_b, scale_v,
                 sem_i0, sem_i1, sem_g0, sem_g1, sem_o0, sem_o1):
    wid = lax.axis_index("s") * NC + lax.axis_index("c")
    ubase = wid * PER_W

    pltpu.sync_copy(scale_hbm, scale_v)
    scale = scale_v[...]

    lane = lax.iota(jnp.int32, 16)
    e_even = lane * 2
    evec_e = (e_even // 8) * STBLK + (e_even % 8) * 128
    e_odd = e_even + 1
    evec_o = (e_odd // 8) * STBLK + (e_odd % 8) * 128

    idx_v = (idx_a, idx_b)
    rows_v = (rows_a, rows_b)
    out_v = (out_a, out_b)
    sem_i = (sem_i0, sem_i1)
    sem_g = (sem_g0, sem_g1)
    sem_o = (sem_o0, sem_o1)

    def flt(ci):
        u0 = ubase + ci * UPC
        return u0 // LT, u0 % LT

    def fire_idx(ci):
        f, lt0 = flt(ci)
        return pltpu.async_copy(
            idx_hbm.at[f, pl.ds(lt0 * 128, CLOOK)], idx_v[ci % 2],
            sem_i[ci % 2])

    def fire_gather(ci):
        return pltpu.async_copy(
            tab_hbm.at[idx_v[ci % 2]], rows_v[ci % 2], sem_g[ci % 2])

    def fire_out(ci):
        f, lt0 = flt(ci)
        return [pltpu.async_copy(
            out_v[ci % 2].at[pl.ds(st * STBLK, STBLK)],
            out_hbm.at[f, st, pl.ds(lt0 * 1024, STBLK)],
            sem_o[ci % 2]) for st in range(4)]

    def compute(ci):
        rows = rows_v[ci % 2]
        out = out_v[ci % 2]

        def row_body(j, _):
            w = plsc.bitcast(rows[j, :], jnp.int32)
            lo = ((w << 16) & jnp.int32(-0x80000000)) | (
                (w << 13) & jnp.int32(0x0FFFE000))
            hi = (w & jnp.int32(-0x80000000)) | (
                lax.shift_right_logical(w, 3) & jnp.int32(0x0FFFE000))
            ev = lax.bitcast_convert_type(lo, jnp.float32) * scale
            od = lax.bitcast_convert_type(hi, jnp.float32) * scale
            base = ((j >> 7) << 10) | (j & 127)
            plsc.store_scatter(out, [base + evec_e], ev)
            plsc.store_scatter(out, [base + evec_o], od)
            return 0

        lax.fori_loop(0, CLOOK, row_body, 0, unroll=2)

    icp = [None] * NCHUNK
    gcp = [None] * NCHUNK
    ocp = [None] * NCHUNK
    icp[0] = fire_idx(0)
    icp[0].wait()
    gcp[0] = fire_gather(0)
    icp[1] = fire_idx(1)
    for ci in range(NCHUNK):
        if ci + 1 < NCHUNK:
            icp[ci + 1].wait()
            gcp[ci + 1] = fire_gather(ci + 1)
        gcp[ci].wait()
        if ci >= 2:
            for cp in ocp[ci - 2]:
                cp.wait()
        compute(ci)
        ocp[ci] = fire_out(ci)
        if ci + 2 < NCHUNK:
            icp[ci + 2] = fire_idx(ci + 2)
    for cp in ocp[NCHUNK - 2] + ocp[NCHUNK - 1]:
        cp.wait()


@jax.jit
def _run(xt, tab, scale16):
    gather = pl.kernel(
        _gather_body,
        mesh=plsc.VectorSubcoreMesh(**_MESH),
        compiler_params=_CP,
        out_type=jax.ShapeDtypeStruct(
            (N_FIELDS, 4, LT * 1024), jnp.float32),
        scratch_types=[
            pltpu.VMEM((CLOOK,), jnp.int32),
            pltpu.VMEM((CLOOK,), jnp.int32),
            pltpu.VMEM((CLOOK, EMBEDDING_DIM), jnp.float16),
            pltpu.VMEM((CLOOK, EMBEDDING_DIM), jnp.float16),
            pltpu.VMEM((4 * STBLK,), jnp.float32),
            pltpu.VMEM((4 * STBLK,), jnp.float32),
            pltpu.VMEM((16,), jnp.float32),
        ] + [pltpu.SemaphoreType.DMA] * 6,
    )
    return gather(xt, tab, scale16)


def kernel(x, weight_quant, c):
    xt = x.T
    scale = jnp.float32(2.0 ** 112) / c
    scale16 = jnp.broadcast_to(scale, (16,))
    out = _run(xt, weight_quant, scale16)
    out = out.reshape(N_FIELDS, 4, LT, 8, 128)
    out = out.transpose(2, 4, 0, 1, 3)
    return out.reshape(BATCH, N_FIELDS, EMBEDDING_DIM)


# speedup vs baseline: 1.8989x; 1.0943x over previous
import jax
import jax.numpy as jnp
from jax import lax
from jax.experimental import pallas as pl
from jax.experimental.pallas import tpu as pltpu
from jax.experimental.pallas import tpu_sc as plsc

NUM_EMBEDDINGS = 1000000
EMBEDDING_DIM = 32
BATCH = 16384
N_FIELDS = 26

NC = 2
NS = 16
NW = NC * NS

LT = BATCH // 128
UNITS = N_FIELDS * LT
PER_W = UNITS // NW
UPC = 8
NCHUNK = PER_W // UPC
CLOOK = UPC * 128

WPR = EMBEDDING_DIM // 2
STBLK = UPC * 1024

_CP = pltpu.CompilerParams(
    needs_layout_passes=False, use_tc_tiling_on_sc=False)
_MESH = dict(core_axis_name="c", subcore_axis_name="s")


def _gather_body(idx_hbm, tab_hbm, scale_hbm, out_hbm,
                 idx_a, idx_b, rows_a, rows_b, out_a, out_b, scale_v,
                 sem_i0, sem_i1, sem_g0, sem_g1, sem_o0, sem_o1):
    wid = lax.axis_index("s") * NC + lax.axis_index("c")
    ubase = wid * PER_W

    pltpu.sync_copy(scale_hbm, scale_v)
    scale = scale_v[...]

    lane = lax.iota(jnp.int32, 16)
    e_even = lane * 2
    evec_e = (e_even // 8) * STBLK + (e_even % 8) * 128
    e_odd = e_even + 1
    evec_o = (e_odd // 8) * STBLK + (e_odd % 8) * 128

    idx_v = (idx_a, idx_b)
    rows_v = (rows_a, rows_b)
    out_v = (out_a, out_b)
    sem_i = (sem_i0, sem_i1)
    sem_g = (sem_g0, sem_g1)
    sem_o = (sem_o0, sem_o1)

    def flt(ci):
        u0 = ubase + ci * UPC
        return u0 // LT, u0 % LT

    def fire_idx(ci):
        f, lt0 = flt(ci)
        return pltpu.async_copy(
            idx_hbm.at[f, pl.ds(lt0 * 128, CLOOK)], idx_v[ci % 2],
            sem_i[ci % 2])

    def fire_gather(ci):
        return pltpu.async_copy(
            tab_hbm.at[idx_v[ci % 2]], rows_v[ci % 2], sem_g[ci % 2])

    def fire_out(ci):
        f, lt0 = flt(ci)
        return [pltpu.async_copy(
            out_v[ci % 2].at[pl.ds(st * STBLK, STBLK)],
            out_hbm.at[f, st, pl.ds(lt0 * 1024, STBLK)],
            sem_o[ci % 2]) for st in range(4)]

    def compute(ci):
        rows = rows_v[ci % 2]
        out = out_v[ci % 2]

        @plsc.parallel_loop(0, CLOOK, unroll=4)
        def row_body(j):
            w = plsc.bitcast(rows[j, :], jnp.int32)
            lo = ((w << 16) & jnp.int32(-0x80000000)) | (
                (w << 13) & jnp.int32(0x0FFFE000))
            hi = (w & jnp.int32(-0x80000000)) | (
                lax.shift_right_logical(w, 3) & jnp.int32(0x0FFFE000))
            ev = lax.bitcast_convert_type(lo, jnp.float32) * scale
            od = lax.bitcast_convert_type(hi, jnp.float32) * scale
            base = ((j >> 7) << 10) | (j & 127)
            plsc.store_scatter(out, [base + evec_e], ev)
            plsc.store_scatter(out, [base + evec_o], od)

    icp = [None] * NCHUNK
    gcp = [None] * NCHUNK
    ocp = [None] * NCHUNK
    icp[0] = fire_idx(0)
    icp[0].wait()
    gcp[0] = fire_gather(0)
    icp[1] = fire_idx(1)
    for ci in range(NCHUNK):
        if ci + 1 < NCHUNK:
            icp[ci + 1].wait()
            gcp[ci + 1] = fire_gather(ci + 1)
        gcp[ci].wait()
        if ci >= 2:
            for cp in ocp[ci - 2]:
                cp.wait()
        compute(ci)
        ocp[ci] = fire_out(ci)
        if ci + 2 < NCHUNK:
            icp[ci + 2] = fire_idx(ci + 2)
    for cp in ocp[NCHUNK - 2] + ocp[NCHUNK - 1]:
        cp.wait()


@jax.jit
def _run(xt, tab, scale16):
    gather = pl.kernel(
        _gather_body,
        mesh=plsc.VectorSubcoreMesh(**_MESH),
        compiler_params=_CP,
        out_type=jax.ShapeDtypeStruct(
            (N_FIELDS, 4, LT * 1024), jnp.float32),
        scratch_types=[
            pltpu.VMEM((CLOOK,), jnp.int32),
            pltpu.VMEM((CLOOK,), jnp.int32),
            pltpu.VMEM((CLOOK, EMBEDDING_DIM), jnp.float16),
            pltpu.VMEM((CLOOK, EMBEDDING_DIM), jnp.float16),
            pltpu.VMEM((4 * STBLK,), jnp.float32),
            pltpu.VMEM((4 * STBLK,), jnp.float32),
            pltpu.VMEM((16,), jnp.float32),
        ] + [pltpu.SemaphoreType.DMA] * 6,
    )
    return gather(xt, tab, scale16)


def kernel(x, weight_quant, c):
    xt = x.T
    scale = jnp.float32(2.0 ** 112) / c
    scale16 = jnp.broadcast_to(scale, (16,))
    out = _run(xt, weight_quant, scale16)
    out = out.reshape(N_FIELDS, 4, LT, 8, 128)
    out = out.transpose(2, 4, 0, 1, 3)
    return out.reshape(BATCH, N_FIELDS, EMBEDDING_DIM)


# speedup vs baseline: 1.9081x; 1.0048x over previous
import jax
import jax.numpy as jnp
from jax import lax
from jax.experimental import pallas as pl
from jax.experimental.pallas import tpu as pltpu
from jax.experimental.pallas import tpu_sc as plsc

NUM_EMBEDDINGS = 1000000
EMBEDDING_DIM = 32
BATCH = 16384
N_FIELDS = 26

NC = 2
NS = 16
NW = NC * NS

LT = BATCH // 128
UNITS = N_FIELDS * LT
PER_W = UNITS // NW
UPC = 8
NCHUNK = PER_W // UPC
CLOOK = UPC * 128

WPR = EMBEDDING_DIM // 2
STBLK = UPC * 1024

_CP = pltpu.CompilerParams(
    needs_layout_passes=False, use_tc_tiling_on_sc=False)
_MESH = dict(core_axis_name="c", subcore_axis_name="s")


def _gather_body(idx_hbm, tab_hbm, scale_hbm, out_hbm,
                 idx_a, idx_b, rows_a, rows_b, out_a, out_b, scale_v,
                 sem_i0, sem_i1, sem_g0, sem_g1, sem_o0, sem_o1):
    wid = lax.axis_index("s") * NC + lax.axis_index("c")
    ubase = wid * PER_W

    pltpu.sync_copy(scale_hbm, scale_v)
    scale = scale_v[...]

    lane = lax.iota(jnp.int32, 16)
    e_even = lane * 2
    evec_e = (e_even // 8) * STBLK + (e_even % 8) * 128
    e_odd = e_even + 1
    evec_o = (e_odd // 8) * STBLK + (e_odd % 8) * 128

    idx_v = (idx_a, idx_b)
    rows_v = (rows_a, rows_b)
    out_v = (out_a, out_b)
    sem_i = (sem_i0, sem_i1)
    sem_g = (sem_g0, sem_g1)
    sem_o = (sem_o0, sem_o1)

    def flt(ci):
        u0 = ubase + ci * UPC
        return u0 // LT, u0 % LT

    def fire_idx(ci):
        f, lt0 = flt(ci)
        return pltpu.async_copy(
            idx_hbm.at[f, pl.ds(lt0 * 128, CLOOK)], idx_v[ci % 2],
            sem_i[ci % 2])

    def fire_gather(ci):
        return pltpu.async_copy(
            tab_hbm.at[idx_v[ci % 2]], rows_v[ci % 2], sem_g[ci % 2])

    def fire_out(ci):
        f, lt0 = flt(ci)
        return [pltpu.async_copy(
            out_v[ci % 2].at[pl.ds(st * STBLK, STBLK)],
            out_hbm.at[f, st, pl.ds(lt0 * 1024, STBLK)],
            sem_o[ci % 2]) for st in range(4)]

    def compute(ci):
        rows = rows_v[ci % 2]
        out = out_v[ci % 2]

        mask = jnp.int32(-1879056384)

        @plsc.parallel_loop(0, CLOOK, unroll=8)
        def row_body(j):
            w = plsc.bitcast(rows[j, :], jnp.int32)
            lo = ((w << 16) >> 3) & mask
            hi = (w >> 3) & mask
            ev = lax.bitcast_convert_type(lo, jnp.float32) * scale
            od = lax.bitcast_convert_type(hi, jnp.float32) * scale
            base = ((j >> 7) << 10) | (j & 127)
            plsc.store_scatter(out, [base + evec_e], ev)
            plsc.store_scatter(out, [base + evec_o], od)

    icp = [None] * NCHUNK
    gcp = [None] * NCHUNK
    ocp = [None] * NCHUNK
    icp[0] = fire_idx(0)
    icp[0].wait()
    gcp[0] = fire_gather(0)
    icp[1] = fire_idx(1)
    for ci in range(NCHUNK):
        if ci + 1 < NCHUNK:
            icp[ci + 1].wait()
            gcp[ci + 1] = fire_gather(ci + 1)
        gcp[ci].wait()
        if ci >= 2:
            for cp in ocp[ci - 2]:
                cp.wait()
        compute(ci)
        ocp[ci] = fire_out(ci)
        if ci + 2 < NCHUNK:
            icp[ci + 2] = fire_idx(ci + 2)
    for cp in ocp[NCHUNK - 2] + ocp[NCHUNK - 1]:
        cp.wait()


@jax.jit
def _run(xt, tab, scale16):
    gather = pl.kernel(
        _gather_body,
        mesh=plsc.VectorSubcoreMesh(**_MESH),
        compiler_params=_CP,
        out_type=jax.ShapeDtypeStruct(
            (N_FIELDS, 4, LT * 1024), jnp.float32),
        scratch_types=[
            pltpu.VMEM((CLOOK,), jnp.int32),
            pltpu.VMEM((CLOOK,), jnp.int32),
            pltpu.VMEM((CLOOK, EMBEDDING_DIM), jnp.float16),
            pltpu.VMEM((CLOOK, EMBEDDING_DIM), jnp.float16),
            pltpu.VMEM((4 * STBLK,), jnp.float32),
            pltpu.VMEM((4 * STBLK,), jnp.float32),
            pltpu.VMEM((16,), jnp.float32),
        ] + [pltpu.SemaphoreType.DMA] * 6,
    )
    return gather(xt, tab, scale16)


def kernel(x, weight_quant, c):
    xt = x.T
    scale = jnp.float32(2.0 ** 112) / c
    scale16 = jnp.broadcast_to(scale, (16,))
    out = _run(xt, weight_quant, scale16)
    out = out.reshape(N_FIELDS, 4, LT, 8, 128)
    out = out.transpose(2, 4, 0, 1, 3)
    return out.reshape(BATCH, N_FIELDS, EMBEDDING_DIM)
